# baseline (speedup 1.0000x reference)
import functools

import jax
import jax.numpy as jnp
from jax import lax
from jax.experimental import pallas as pl
from jax.experimental.pallas import tpu as pltpu
from jax.experimental.pallas import tpu_sc as plsc

NN = 10000
NE = 320000
DIM = 128
NC = 2
NS = 16
EC = 80
EPT = NE // (NC * NS)
AEC = 64
PPT = 240
ACPT = (EPT + PPT) // AEC
SG = 8
NSG = ACPT // SG
NBUF = 4
NNP = NN + 16
DCPT = 2 * NE // (NC * NS) // EC
DGB = 10
SP = 624
LAST = NN - (NS - 1) * SP


def _sc_mesh():
    return plsc.VectorSubcoreMesh(
        core_axis_name="c", subcore_axis_name="s",
        num_cores=NC, num_subcores=NS)


def _zero_acc(s, zbuf, acc):
    def zc(j, _):
        pltpu.sync_copy(zbuf, acc.at[pl.ds(s * SP + j * 16, 16), :])
        return 0
    lax.fori_loop(0, SP // 16, zc, 0)

    @pl.when(s == NS - 1)
    def _():
        for j in range(SP // 16, LAST // 16):
            pltpu.sync_copy(zbuf, acc.at[pl.ds(s * SP + j * 16, 16), :])


def _copy_out_rows(s, acc, out_slice_fn):
    @pl.when(s < NS - 1)
    def _():
        pltpu.sync_copy(acc.at[pl.ds(s * SP, SP), :],
                        out_slice_fn(s * SP, SP))

    @pl.when(s == NS - 1)
    def _():
        pltpu.sync_copy(acc.at[pl.ds((NS - 1) * SP, LAST), :],
                        out_slice_fn((NS - 1) * SP, LAST))


def _make_deg(dw=DIM):
    @functools.partial(
        pl.kernel,
        out_type=jax.ShapeDtypeStruct((NC, NN, dw), jnp.float32),
        mesh=_sc_mesh(),
        scratch_types=[
            pltpu.VMEM_SHARED((NN, dw), jnp.float32),
            pltpu.VMEM((16, dw), jnp.float32),
            pltpu.VMEM((EC, dw), jnp.float32),
            pltpu.VMEM((DCPT, EC), jnp.int32),
            pltpu.SemaphoreType.DMA,
        ],
    )
    def deg(echunks_hbm, out_hbm, acc, zbuf, ones_v, idx_all, ssem):
        c = lax.axis_index("c")
        s = lax.axis_index("s")
        w = c * NS + s
        z16 = jnp.zeros((16,), jnp.float32)
        o16 = jnp.ones((16,), jnp.float32)

        def zrow(r, _):
            for k in range(dw // 16):
                zbuf[r, pl.ds(k * 16, 16)] = z16
            return 0
        lax.fori_loop(0, 16, zrow, 0)

        def orow(r, _):
            for k in range(dw // 16):
                ones_v[r, pl.ds(k * 16, 16)] = o16
            return 0
        lax.fori_loop(0, EC, orow, 0)

        pltpu.sync_copy(echunks_hbm.at[w], idx_all)
        _zero_acc(s, zbuf, acc)
        plsc.subcore_barrier()

        ncs = DCPT // DGB

        def group(g, _):
            descs = []
            for b in range(DGB):
                ci = g * DGB + b
                descs.append(pltpu.async_copy(
                    ones_v, acc.at[idx_all.at[ci]], ssem, add=True))
            for d in descs:
                d.wait()
            return 0
        lax.fori_loop(0, ncs, group, 0)
        plsc.subcore_barrier()

        _copy_out_rows(s, acc,
                       lambda st, sz: out_hbm.at[c, pl.ds(st, sz), :])
    return deg


def _make_agg():
    @functools.partial(
        pl.kernel,
        out_type=jax.ShapeDtypeStruct((NC, NN, DIM), jnp.float32),
        mesh=_sc_mesh(),
        scratch_types=[
            pltpu.VMEM_SHARED((NNP, DIM), jnp.float32),
            pltpu.VMEM((16, DIM), jnp.float32),
            pltpu.VMEM((SG, AEC), jnp.int32),
            pltpu.VMEM((SG, AEC), jnp.int32),
            [pltpu.VMEM((AEC, DIM), jnp.float32)] * NBUF,
            pltpu.SemaphoreType.DMA,
            pltpu.SemaphoreType.DMA,
        ],
    )
    def agg(h_hbm, srcc_hbm, dstc_hbm, out_hbm, acc, zbuf, sbuf, dbuf,
            rows, gsem, ssem):
        c = lax.axis_index("c")
        s = lax.axis_index("s")
        w = c * NS + s
        z16 = jnp.zeros((16,), jnp.float32)

        def zrow(r, _):
            for k in range(DIM // 16):
                zbuf[r, pl.ds(k * 16, 16)] = z16
            return 0
        lax.fori_loop(0, 16, zrow, 0)

        _zero_acc(s, zbuf, acc)
        plsc.subcore_barrier()

        def sgroup(sg, _):
            pltpu.sync_copy(srcc_hbm.at[w, pl.ds(sg * SG, SG), :], sbuf)
            pltpu.sync_copy(dstc_hbm.at[w, pl.ds(sg * SG, SG), :], dbuf)
            for b in range(NBUF):
                pltpu.async_copy(h_hbm.at[sbuf.at[b]], rows[b], gsem)
            for gb in range(SG // NBUF):
                sdescs = []
                for b in range(NBUF):
                    cl = gb * NBUF + b
                    pltpu.make_async_copy(
                        h_hbm.at[sbuf.at[cl]], rows[b], gsem).wait()
                    sdescs.append(pltpu.async_copy(
                        rows[b], acc.at[dbuf.at[cl]], ssem, add=True))
                for b in range(NBUF):
                    sdescs[b].wait()
                    cl2 = (gb + 1) * NBUF + b
                    if cl2 < SG:
                        pltpu.async_copy(
                            h_hbm.at[sbuf.at[cl2]], rows[b], gsem)
            return 0
        lax.fori_loop(0, NSG, sgroup, 0)
        plsc.subcore_barrier()

        _copy_out_rows(s, acc,
                       lambda st, sz: out_hbm.at[c, pl.ds(st, sz), :])
    return agg


_ROWS_BLK = 1000


def _norm_col(degs, which):
    return lax.rsqrt(jnp.maximum(degs[which, :, 0:1], 1.0))


def _pre_body(x_ref, degs_ref, o_ref):
    o_ref[...] = x_ref[...] * _norm_col(degs_ref[...], 0)


def _pre(x, degs):
    grid = NN // _ROWS_BLK
    return pl.pallas_call(
        _pre_body,
        grid=(grid,),
        in_specs=[
            pl.BlockSpec((_ROWS_BLK, DIM), lambda i: (i, 0)),
            pl.BlockSpec((NC, _ROWS_BLK, DIM), lambda i: (0, i, 0)),
        ],
        out_specs=pl.BlockSpec((_ROWS_BLK, DIM), lambda i: (i, 0)),
        out_shape=jax.ShapeDtypeStruct((NN, DIM), jnp.float32),
    )(x, degs)


def _layer_body(scale_out, p_ref, degs_ref, w_ref, b_ref, o_ref):
    p = p_ref[0] + p_ref[1]
    degs = degs_ref[...]
    h = jnp.dot(p * _norm_col(degs, 1), w_ref[...],
                preferred_element_type=jnp.float32)
    h = jnp.maximum(h + b_ref[...], 0.0)
    if scale_out:
        h = h * _norm_col(degs, 0)
    o_ref[...] = h


def _layer(parts, degs, w, b, scale_out):
    grid = NN // _ROWS_BLK
    return pl.pallas_call(
        functools.partial(_layer_body, scale_out),
        grid=(grid,),
        in_specs=[
            pl.BlockSpec((NC, _ROWS_BLK, DIM), lambda i: (0, i, 0)),
            pl.BlockSpec((NC, _ROWS_BLK, DIM), lambda i: (0, i, 0)),
            pl.BlockSpec((DIM, DIM), lambda i: (0, 0)),
            pl.BlockSpec((1, DIM), lambda i: (0, 0)),
        ],
        out_specs=pl.BlockSpec((_ROWS_BLK, DIM), lambda i: (i, 0)),
        out_shape=jax.ShapeDtypeStruct((NN, DIM), jnp.float32),
    )(parts, degs, w, b)


def _head_body(h_ref, w1_ref, b1_ref, w2_ref, b2_ref, o_ref):
    hg = jnp.sum(h_ref[...], axis=0, keepdims=True) * (1.0 / NN)
    a = jnp.dot(hg, w1_ref[...], preferred_element_type=jnp.float32)
    a = jnp.maximum(a + b1_ref[...], 0.0)
    o = jnp.dot(a, w2_ref[...], preferred_element_type=jnp.float32)
    o_ref[...] = o + b2_ref[...]


def _head(h, w1, b1, w2, b2):
    n_cls = w2.shape[1]
    return pl.pallas_call(
        _head_body,
        out_shape=jax.ShapeDtypeStruct((1, n_cls), jnp.float32),
    )(h, w1, b1, w2, b2)


def kernel(x, edge_index, W1, b1, W2, b2, W3, b3, W4, b4, Wc1, bc1, Wc2, bc2):
    ei = edge_index.astype(jnp.int32)
    nw = NC * NS
    pad_src = (jnp.arange(nw * PPT, dtype=jnp.int32) % NN).reshape(nw, PPT)
    pad_dst = NN + (jnp.arange(nw * PPT, dtype=jnp.int32) % 16).reshape(nw, PPT)
    srcc = jnp.concatenate(
        [ei[0].reshape(nw, EPT), pad_src], axis=1).reshape(nw, ACPT, AEC)
    dstc = jnp.concatenate(
        [ei[1].reshape(nw, EPT), pad_dst], axis=1).reshape(nw, ACPT, AEC)
    echunks = ei.reshape(nw, DCPT, EC)

    deg_fn = _make_deg()
    agg_fn = _make_agg()

    degs = deg_fn(echunks)
    h = _pre(x, degs)
    layers = [(W1, b1, True), (W2, b2, True), (W3, b3, True), (W4, b4, False)]
    for w, b, scale_out in layers:
        parts = agg_fn(h, srcc, dstc)
        h = _layer(parts, degs, w, b.reshape(1, DIM), scale_out)
    return _head(h, Wc1, bc1.reshape(1, -1), Wc2, bc2.reshape(1, -1))


# speedup vs baseline: 1.0773x; 1.0773x over previous
import functools

import jax
import jax.numpy as jnp
from jax import lax
from jax.experimental import pallas as pl
from jax.experimental.pallas import tpu as pltpu
from jax.experimental.pallas import tpu_sc as plsc

NN = 10000
NE = 320000
DIM = 128
NC = 2
NS = 16
EC = 80
EPT = NE // (NC * NS)
AEC = 128
PPT = 240
ACPT = (EPT + PPT) // AEC
SG = 16
NSG = ACPT // SG
NBUF = 2
NNP = NN + 16
DCPT = 2 * NE // (NC * NS) // EC
DGB = 10
SP = 624
LAST = NN - (NS - 1) * SP


def _sc_mesh():
    return plsc.VectorSubcoreMesh(
        core_axis_name="c", subcore_axis_name="s",
        num_cores=NC, num_subcores=NS)


def _zero_acc(s, zbuf, acc):
    zr = zbuf.shape[0]

    def zc(j, _):
        pltpu.sync_copy(zbuf, acc.at[pl.ds(s * SP + j * zr, zr), :])
        return 0
    lax.fori_loop(0, SP // zr, zc, 0)

    @pl.when(s == NS - 1)
    def _():
        for j in range(SP // zr, LAST // zr):
            pltpu.sync_copy(zbuf, acc.at[pl.ds(s * SP + j * zr, zr), :])


def _copy_out_rows(s, acc, out_slice_fn):
    @pl.when(s < NS - 1)
    def _():
        pltpu.sync_copy(acc.at[pl.ds(s * SP, SP), :],
                        out_slice_fn(s * SP, SP))

    @pl.when(s == NS - 1)
    def _():
        pltpu.sync_copy(acc.at[pl.ds((NS - 1) * SP, LAST), :],
                        out_slice_fn((NS - 1) * SP, LAST))


def _make_deg(dw=DIM):
    @functools.partial(
        pl.kernel,
        out_type=jax.ShapeDtypeStruct((NC, NN, dw), jnp.float32),
        mesh=_sc_mesh(),
        scratch_types=[
            pltpu.VMEM_SHARED((NN, dw), jnp.float32),
            pltpu.VMEM((16, dw), jnp.float32),
            pltpu.VMEM((EC, dw), jnp.float32),
            pltpu.VMEM((DCPT, EC), jnp.int32),
            pltpu.SemaphoreType.DMA,
        ],
    )
    def deg(echunks_hbm, out_hbm, acc, zbuf, ones_v, idx_all, ssem):
        c = lax.axis_index("c")
        s = lax.axis_index("s")
        w = c * NS + s
        z16 = jnp.zeros((16,), jnp.float32)
        o16 = jnp.ones((16,), jnp.float32)

        def zrow(r, _):
            for k in range(dw // 16):
                zbuf[r, pl.ds(k * 16, 16)] = z16
            return 0
        lax.fori_loop(0, 16, zrow, 0)

        def orow(r, _):
            for k in range(dw // 16):
                ones_v[r, pl.ds(k * 16, 16)] = o16
            return 0
        lax.fori_loop(0, EC, orow, 0)

        pltpu.sync_copy(echunks_hbm.at[w], idx_all)
        _zero_acc(s, zbuf, acc)
        plsc.subcore_barrier()

        ncs = DCPT // DGB

        def group(g, _):
            descs = []
            for b in range(DGB):
                ci = g * DGB + b
                descs.append(pltpu.async_copy(
                    ones_v, acc.at[idx_all.at[ci]], ssem, add=True))
            for d in descs:
                d.wait()
            return 0
        lax.fori_loop(0, ncs, group, 0)
        plsc.subcore_barrier()

        _copy_out_rows(s, acc,
                       lambda st, sz: out_hbm.at[c, pl.ds(st, sz), :])
    return deg


def _make_agg():
    @functools.partial(
        pl.kernel,
        out_type=jax.ShapeDtypeStruct((NC, NN, DIM), jnp.float32),
        mesh=_sc_mesh(),
        scratch_types=[
            pltpu.VMEM_SHARED((NNP, DIM), jnp.float32),
            pltpu.VMEM((8, DIM), jnp.float32),
            pltpu.VMEM((SG, AEC), jnp.int32),
            pltpu.VMEM((SG, AEC), jnp.int32),
            [pltpu.VMEM((AEC, DIM), jnp.float32)] * NBUF,
            pltpu.SemaphoreType.DMA,
            pltpu.SemaphoreType.DMA,
        ],
    )
    def agg(h_hbm, srcc_hbm, dstc_hbm, out_hbm, acc, zbuf, sbuf, dbuf,
            rows, gsem, ssem):
        c = lax.axis_index("c")
        s = lax.axis_index("s")
        w = c * NS + s
        z16 = jnp.zeros((16,), jnp.float32)

        def zrow(r, _):
            for k in range(DIM // 16):
                zbuf[r, pl.ds(k * 16, 16)] = z16
            return 0
        lax.fori_loop(0, 8, zrow, 0)

        _zero_acc(s, zbuf, acc)
        plsc.subcore_barrier()

        def sgroup(sg, _):
            pltpu.sync_copy(srcc_hbm.at[w, pl.ds(sg * SG, SG), :], sbuf)
            pltpu.sync_copy(dstc_hbm.at[w, pl.ds(sg * SG, SG), :], dbuf)
            for b in range(NBUF):
                pltpu.async_copy(h_hbm.at[sbuf.at[b]], rows[b], gsem)
            for gb in range(SG // NBUF):
                sdescs = []
                for b in range(NBUF):
                    cl = gb * NBUF + b
                    pltpu.make_async_copy(
                        h_hbm.at[sbuf.at[cl]], rows[b], gsem).wait()
                    sdescs.append(pltpu.async_copy(
                        rows[b], acc.at[dbuf.at[cl]], ssem, add=True))
                for b in range(NBUF):
                    sdescs[b].wait()
                    cl2 = (gb + 1) * NBUF + b
                    if cl2 < SG:
                        pltpu.async_copy(
                            h_hbm.at[sbuf.at[cl2]], rows[b], gsem)
            return 0
        lax.fori_loop(0, NSG, sgroup, 0)
        plsc.subcore_barrier()

        _copy_out_rows(s, acc,
                       lambda st, sz: out_hbm.at[c, pl.ds(st, sz), :])
    return agg


_ROWS_BLK = 1000


def _norm_col(degs, which):
    return lax.rsqrt(jnp.maximum(degs[which, :, 0:1], 1.0))


def _pre_body(x_ref, degs_ref, o_ref):
    o_ref[...] = x_ref[...] * _norm_col(degs_ref[...], 0)


def _pre(x, degs):
    grid = NN // _ROWS_BLK
    return pl.pallas_call(
        _pre_body,
        grid=(grid,),
        in_specs=[
            pl.BlockSpec((_ROWS_BLK, DIM), lambda i: (i, 0)),
            pl.BlockSpec((NC, _ROWS_BLK, DIM), lambda i: (0, i, 0)),
        ],
        out_specs=pl.BlockSpec((_ROWS_BLK, DIM), lambda i: (i, 0)),
        out_shape=jax.ShapeDtypeStruct((NN, DIM), jnp.float32),
    )(x, degs)


def _layer_body(scale_out, p_ref, degs_ref, w_ref, b_ref, o_ref):
    p = p_ref[0] + p_ref[1]
    degs = degs_ref[...]
    h = jnp.dot(p * _norm_col(degs, 1), w_ref[...],
                preferred_element_type=jnp.float32)
    h = jnp.maximum(h + b_ref[...], 0.0)
    if scale_out:
        h = h * _norm_col(degs, 0)
    o_ref[...] = h


def _layer(parts, degs, w, b, scale_out):
    grid = NN // _ROWS_BLK
    return pl.pallas_call(
        functools.partial(_layer_body, scale_out),
        grid=(grid,),
        in_specs=[
            pl.BlockSpec((NC, _ROWS_BLK, DIM), lambda i: (0, i, 0)),
            pl.BlockSpec((NC, _ROWS_BLK, DIM), lambda i: (0, i, 0)),
            pl.BlockSpec((DIM, DIM), lambda i: (0, 0)),
            pl.BlockSpec((1, DIM), lambda i: (0, 0)),
        ],
        out_specs=pl.BlockSpec((_ROWS_BLK, DIM), lambda i: (i, 0)),
        out_shape=jax.ShapeDtypeStruct((NN, DIM), jnp.float32),
    )(parts, degs, w, b)


def _head_body(h_ref, w1_ref, b1_ref, w2_ref, b2_ref, o_ref):
    hg = jnp.sum(h_ref[...], axis=0, keepdims=True) * (1.0 / NN)
    a = jnp.dot(hg, w1_ref[...], preferred_element_type=jnp.float32)
    a = jnp.maximum(a + b1_ref[...], 0.0)
    o = jnp.dot(a, w2_ref[...], preferred_element_type=jnp.float32)
    o_ref[...] = o + b2_ref[...]


def _head(h, w1, b1, w2, b2):
    n_cls = w2.shape[1]
    return pl.pallas_call(
        _head_body,
        out_shape=jax.ShapeDtypeStruct((1, n_cls), jnp.float32),
    )(h, w1, b1, w2, b2)


def kernel(x, edge_index, W1, b1, W2, b2, W3, b3, W4, b4, Wc1, bc1, Wc2, bc2):
    ei = edge_index.astype(jnp.int32)
    nw = NC * NS
    pad_src = (jnp.arange(nw * PPT, dtype=jnp.int32) % NN).reshape(nw, PPT)
    pad_dst = NN + (jnp.arange(nw * PPT, dtype=jnp.int32) % 16).reshape(nw, PPT)
    srcc = jnp.concatenate(
        [ei[0].reshape(nw, EPT), pad_src], axis=1).reshape(nw, ACPT, AEC)
    dstc = jnp.concatenate(
        [ei[1].reshape(nw, EPT), pad_dst], axis=1).reshape(nw, ACPT, AEC)
    echunks = ei.reshape(nw, DCPT, EC)

    deg_fn = _make_deg()
    agg_fn = _make_agg()

    degs = deg_fn(echunks)
    h = _pre(x, degs)
    layers = [(W1, b1, True), (W2, b2, True), (W3, b3, True), (W4, b4, False)]
    for w, b, scale_out in layers:
        parts = agg_fn(h, srcc, dstc)
        h = _layer(parts, degs, w, b.reshape(1, DIM), scale_out)
    return _head(h, Wc1, bc1.reshape(1, -1), Wc2, bc2.reshape(1, -1))
